# baseline (speedup 1.0000x reference)
import functools

import jax
import jax.numpy as jnp
from jax import lax
from jax.experimental import pallas as pl
from jax.experimental.pallas import tpu as pltpu
from jax.experimental.pallas import tpu_sc as plsc

VOCAB = 1000000
EMBED_DIM = 32
BATCH = 4096
SEQ = 200

_INFO = plsc.get_sparse_core_info()
NC, NS, NL = _INFO.num_cores, _INFO.num_subcores, _INFO.num_lanes
NW = NC * NS
BT = BATCH // NW
LG = 4
N_GROUPS = SEQ // LG
NRB = 3
NTB = 2
UNROLL = NRB * NTB
N_MAIN = (N_GROUPS - 2) // UNROLL
DHI = EMBED_DIM // 8
INNER = 8 * BT


def _body(x_h, emb_h, out_h, x_v, rows0, rows1, rows2, tb0, tb1,
          gsem0, gsem1, gsem2, wsem0, wsem1):
    rows = (rows0, rows1, rows2)
    gsem = (gsem0, gsem1, gsem2)
    tb = (tb0, tb1)
    wsem = (wsem0, wsem1)

    wid = lax.axis_index("s") * NC + lax.axis_index("c")
    pltpu.sync_copy(x_h.at[:, wid], x_v)

    iota = lax.iota(jnp.int32, NL)
    li_vecs = [jnp.broadcast_to(jnp.int32(li), (NL,)) for li in range(LG)]

    def fire(g, r, sem):
        for li in range(LG):
            l = g * LG + li
            pltpu.async_copy(
                emb_h.at[x_v.at[l // 8, l % 8]], r.at[li], sem)

    def drain_gather(r, sem):
        for li in range(LG):
            pltpu.make_async_copy(
                emb_h.at[pl.ds(0, BT)], r.at[li], sem).wait()

    def transpose_group(r, t):
        @plsc.parallel_loop(0, 16, unroll=2)
        def tbody(tt):
            p = (iota + tt) & 15
            pats = []
            for dblk in range(2):
                patd = p + dblk * 16
                pats.append((patd, patd >> 3, (patd & 7) << 7))
            for li in range(LG):
                staged = []
                for c in range(BT // NL):
                    b = iota + c * NL
                    for dblk in range(2):
                        patd, pathi, patflat = pats[dblk]
                        staged.append(
                            (plsc.load_gather(r, [li_vecs[li], b, patd]),
                             pathi, patflat + b))
                for vals, pathi, flat in staged:
                    plsc.store_scatter(t, [li_vecs[li], pathi, flat], vals)

    def writeback(g, t, sem):
        pltpu.async_copy(
            t, out_h.at[pl.ds(g * LG, LG), :, pl.ds(wid * INNER, INNER)],
            sem)

    def wait_writeback(t, sem):
        pltpu.make_async_copy(
            t, out_h.at[pl.ds(0, LG), :, pl.ds(0, INNER)], sem).wait()

    def handle_group(g, ri, ti, guard_wb, guard_fire):
        drain_gather(rows[ri], gsem[ri])
        if guard_wb is None:
            wait_writeback(tb[ti], wsem[ti])
        elif guard_wb is not False:
            @pl.when(guard_wb)
            def _():
                wait_writeback(tb[ti], wsem[ti])
        transpose_group(rows[ri], tb[ti])
        writeback(g, tb[ti], wsem[ti])
        if guard_fire is None:
            fire(g + NRB, rows[ri], gsem[ri])
        elif guard_fire is not False:
            @pl.when(guard_fire)
            def _():
                fire(g + NRB, rows[ri], gsem[ri])

    for g0 in range(NRB):
        fire(g0, rows[g0], gsem[g0])

    def body(i, carry):
        gb = UNROLL * i
        for j in range(UNROLL):
            g = gb + j
            handle_group(g, j % NRB, j % NTB,
                         guard_wb=(g >= NTB) if j < NTB else None,
                         guard_fire=(g + NRB < N_GROUPS))
        return carry

    lax.fori_loop(0, N_MAIN, body, 0)

    for g in range(N_MAIN * UNROLL, N_GROUPS):
        handle_group(g, g % NRB, g % NTB, guard_wb=None, guard_fire=False)

    wait_writeback(tb[0], wsem[0])
    wait_writeback(tb[1], wsem[1])


DP_W = 512
DP_NB = VOCAB // DP_W
DP_TAIL = VOCAB - DP_NB * DP_W
DP_SLOTS = 3


def _fmt_block(iota, bufin, bufout, ncols):
    @plsc.parallel_loop(0, 16, unroll=2)
    def tbody(tt):
        p = (iota + tt) & 15
        for dblk in range(2):
            patd = p + dblk * 16
            base = (iota << 5) + patd
            h0 = base >> 7
            l0 = base & 127
            for c in range(ncols // 16):
                vals = plsc.load_gather(bufin, [patd, iota + c * 16])
                plsc.store_scatter(bufout, [h0 + 4 * c, l0], vals)


def _fmt_body(emb_t, tail_h, out_l, bi0, bi1, bi2, bo0, bo1, bo2,
              is0, is1, is2, os0, os1, os2):
    bin_ = (bi0, bi1, bi2)
    bout = (bo0, bo1, bo2)
    isem = (is0, is1, is2)
    osem = (os0, os1, os2)
    wid = lax.axis_index("s") * NC + lax.axis_index("c")
    iota = lax.iota(jnp.int32, NL)

    def fire_in(b, j):
        @pl.when(b < DP_NB)
        def _():
            pltpu.async_copy(emb_t.at[:, pl.ds(b * DP_W, DP_W)], bin_[j],
                             isem[j])

    def drain_in(j):
        pltpu.make_async_copy(emb_t.at[:, pl.ds(0, DP_W)], bin_[j],
                              isem[j]).wait()

    def wait_out(j):
        pltpu.make_async_copy(bout[j], out_l.at[pl.ds(0, DP_W // 4), :],
                              osem[j]).wait()

    for j in range(DP_SLOTS):
        fire_in(wid + NW * j, j)

    def body(k, carry):
        for j in range(DP_SLOTS):
            kk = DP_SLOTS * k + j
            b = wid + NW * kk

            @pl.when(b < DP_NB)
            def _():
                drain_in(j)

                @pl.when(kk >= DP_SLOTS)
                def _():
                    wait_out(j)

                _fmt_block(iota, bin_[j], bout[j], DP_W)
                pltpu.async_copy(
                    bout[j], out_l.at[pl.ds(b * (DP_W // 4), DP_W // 4), :],
                    osem[j])
                fire_in(b + NW * DP_SLOTS, j)
        return carry

    lax.fori_loop(0, 21, body, 0)

    for j in range(DP_SLOTS):
        wait_out(j)

    @pl.when(wid == 0)
    def _():
        pltpu.sync_copy(tail_h, bo0.at[pl.ds(0, DP_TAIL // 4), :])
        pltpu.sync_copy(bo0.at[pl.ds(0, DP_TAIL // 4), :],
                        out_l.at[pl.ds(DP_NB * DP_W // 4, DP_TAIL // 4), :])


def _format_table(emb):
    mesh = plsc.VectorSubcoreMesh(core_axis_name="c", subcore_axis_name="s")
    sc = [pltpu.VMEM((EMBED_DIM, DP_W), jnp.float32) for _ in range(DP_SLOTS)]
    sc += [pltpu.VMEM((DP_W // 4, 128), jnp.float32) for _ in range(DP_SLOTS)]
    sc += [pltpu.SemaphoreType.DMA for _ in range(2 * DP_SLOTS)]
    return pl.kernel(
        _fmt_body,
        mesh=mesh,
        out_type=jax.ShapeDtypeStruct((VOCAB // 4, 128), jnp.float32),
        scratch_types=sc,
        compiler_params=pltpu.CompilerParams(
            use_tc_tiling_on_sc=True, needs_layout_passes=False),
    )(jnp.transpose(emb),
      emb[DP_NB * DP_W:, :].reshape(DP_TAIL // 4, 128))


@functools.partial(jax.jit, static_argnames=())
def kernel(x, hidden, emb):
    del hidden
    x5 = jnp.transpose(x, (1, 0)).reshape(SEQ // 8, 8, NW, BT)
    x5 = jnp.transpose(x5, (0, 2, 1, 3))

    embl = _format_table(emb).reshape(VOCAB, EMBED_DIM)

    mesh = plsc.VectorSubcoreMesh(core_axis_name="c", subcore_axis_name="s")
    y3 = pl.kernel(
        _body,
        mesh=mesh,
        out_type=jax.ShapeDtypeStruct((SEQ, DHI, NW * INNER), jnp.float32),
        scratch_types=[
            pltpu.VMEM((SEQ // 8, 8, BT), jnp.int32),
            pltpu.VMEM((LG, BT, EMBED_DIM), jnp.float32),
            pltpu.VMEM((LG, BT, EMBED_DIM), jnp.float32),
            pltpu.VMEM((LG, BT, EMBED_DIM), jnp.float32),
            pltpu.VMEM((LG, DHI, INNER), jnp.float32),
            pltpu.VMEM((LG, DHI, INNER), jnp.float32),
            pltpu.SemaphoreType.DMA,
            pltpu.SemaphoreType.DMA,
            pltpu.SemaphoreType.DMA,
            pltpu.SemaphoreType.DMA,
            pltpu.SemaphoreType.DMA,
        ],
        compiler_params=pltpu.CompilerParams(
            use_tc_tiling_on_sc=False, needs_layout_passes=False),
    )(x5, embl)

    y5 = y3.reshape(SEQ, DHI, NW, 8, BT)
    return jnp.transpose(y5, (0, 2, 4, 1, 3)).reshape(SEQ, BATCH, EMBED_DIM)


# speedup vs baseline: 1.0562x; 1.0562x over previous
import functools

import jax
import jax.numpy as jnp
from jax import lax
from jax.experimental import pallas as pl
from jax.experimental.pallas import tpu as pltpu
from jax.experimental.pallas import tpu_sc as plsc

VOCAB = 1000000
EMBED_DIM = 32
BATCH = 4096
SEQ = 200

_INFO = plsc.get_sparse_core_info()
NC, NS, NL = _INFO.num_cores, _INFO.num_subcores, _INFO.num_lanes
NW = NC * NS
BT = BATCH // NW
LG = 4
N_GROUPS = SEQ // LG
NRB = 3
NTB = 2
UNROLL = NRB * NTB
N_MAIN = (N_GROUPS - 2) // UNROLL
DHI = EMBED_DIM // 8
INNER = 8 * BT


def _body(x_h, emb_h, out_h, x_v, rows0, rows1, rows2, tb0, tb1,
          gsem0, gsem1, gsem2, wsem0, wsem1):
    rows = (rows0, rows1, rows2)
    gsem = (gsem0, gsem1, gsem2)
    tb = (tb0, tb1)
    wsem = (wsem0, wsem1)

    wid = lax.axis_index("s") * NC + lax.axis_index("c")
    pltpu.sync_copy(x_h.at[:, wid], x_v)

    iota = lax.iota(jnp.int32, NL)
    li_vecs = [jnp.broadcast_to(jnp.int32(li), (NL,)) for li in range(LG)]

    def fire(g, r, sem):
        for li in range(LG):
            l = g * LG + li
            pltpu.async_copy(
                emb_h.at[x_v.at[l // 8, l % 8]], r.at[li], sem)

    def drain_gather(r, sem):
        for li in range(LG):
            pltpu.make_async_copy(
                emb_h.at[pl.ds(0, BT)], r.at[li], sem).wait()

    def transpose_group(r, t):
        @plsc.parallel_loop(0, 16, unroll=2)
        def tbody(tt):
            p = (iota + tt) & 15
            pats = []
            for dblk in range(2):
                patd = p + dblk * 16
                pats.append((patd, patd >> 3, (patd & 7) << 7))
            for li in range(LG):
                staged = []
                for c in range(BT // NL):
                    b = iota + c * NL
                    for dblk in range(2):
                        patd, pathi, patflat = pats[dblk]
                        staged.append(
                            (plsc.load_gather(r, [li_vecs[li], b, patd]),
                             pathi, patflat + b))
                for vals, pathi, flat in staged:
                    plsc.store_scatter(t, [li_vecs[li], pathi, flat], vals)

    def writeback(g, t, sem):
        pltpu.async_copy(
            t, out_h.at[pl.ds(g * LG, LG), :, pl.ds(wid * INNER, INNER)],
            sem)

    def wait_writeback(t, sem):
        pltpu.make_async_copy(
            t, out_h.at[pl.ds(0, LG), :, pl.ds(0, INNER)], sem).wait()

    def handle_group(g, ri, ti, guard_wb, guard_fire):
        drain_gather(rows[ri], gsem[ri])
        if guard_wb is None:
            wait_writeback(tb[ti], wsem[ti])
        elif guard_wb is not False:
            @pl.when(guard_wb)
            def _():
                wait_writeback(tb[ti], wsem[ti])
        transpose_group(rows[ri], tb[ti])
        writeback(g, tb[ti], wsem[ti])
        if guard_fire is None:
            fire(g + NRB, rows[ri], gsem[ri])
        elif guard_fire is not False:
            @pl.when(guard_fire)
            def _():
                fire(g + NRB, rows[ri], gsem[ri])

    for g0 in range(NRB):
        fire(g0, rows[g0], gsem[g0])

    def body(i, carry):
        gb = UNROLL * i
        for j in range(UNROLL):
            g = gb + j
            handle_group(g, j % NRB, j % NTB,
                         guard_wb=(g >= NTB) if j < NTB else None,
                         guard_fire=(g + NRB < N_GROUPS))
        return carry

    lax.fori_loop(0, N_MAIN, body, 0)

    for g in range(N_MAIN * UNROLL, N_GROUPS):
        handle_group(g, g % NRB, g % NTB, guard_wb=None, guard_fire=False)

    wait_writeback(tb[0], wsem[0])
    wait_writeback(tb[1], wsem[1])


DP_W = 512
DP_NB = VOCAB // DP_W
DP_TAIL = VOCAB - DP_NB * DP_W
DP_SLOTS = 2


def _fmt_block(iota, bufin, bufout, ncols):
    @plsc.parallel_loop(0, 16, unroll=2)
    def tbody(tt):
        p = (iota + tt) & 15
        for c in range(ncols // 16):
            rl32 = (iota + c * 16) << 5
            for dblk in range(2):
                patd = p + dblk * 16
                vals = plsc.load_gather(bufin, [patd, iota + c * 16])
                flat = rl32 + patd
                plsc.store_scatter(bufout, [flat >> 7, flat & 127], vals)


def _fmt_body(emb_t, tail_h, out_l, bi0, bi1, bo0, bo1, is0, is1, os0, os1):
    bin_ = (bi0, bi1)
    bout = (bo0, bo1)
    isem = (is0, is1)
    osem = (os0, os1)
    wid = lax.axis_index("s") * NC + lax.axis_index("c")
    iota = lax.iota(jnp.int32, NL)

    def fire_in(b, j):
        @pl.when(b < DP_NB)
        def _():
            pltpu.async_copy(emb_t.at[:, pl.ds(b * DP_W, DP_W)], bin_[j],
                             isem[j])

    def drain_in(j):
        pltpu.make_async_copy(emb_t.at[:, pl.ds(0, DP_W)], bin_[j],
                              isem[j]).wait()

    def wait_out(j):
        pltpu.make_async_copy(bout[j], out_l.at[pl.ds(0, DP_W // 4), :],
                              osem[j]).wait()

    fire_in(wid, 0)
    fire_in(wid + NW, 1)

    def body(k, carry):
        for j in range(DP_SLOTS):
            kk = DP_SLOTS * k + j
            b = wid + NW * kk

            @pl.when(b < DP_NB)
            def _():
                drain_in(j)

                @pl.when(kk >= DP_SLOTS)
                def _():
                    wait_out(j)

                _fmt_block(iota, bin_[j], bout[j], DP_W)
                pltpu.async_copy(
                    bout[j], out_l.at[pl.ds(b * (DP_W // 4), DP_W // 4), :],
                    osem[j])
                fire_in(b + NW * DP_SLOTS, j)
        return carry

    lax.fori_loop(0, 31, body, 0)

    for j in range(DP_SLOTS):
        wait_out(j)

    @pl.when(wid == 0)
    def _():
        pltpu.sync_copy(tail_h, bo0.at[pl.ds(0, DP_TAIL // 4), :])
        pltpu.sync_copy(bo0.at[pl.ds(0, DP_TAIL // 4), :],
                        out_l.at[pl.ds(DP_NB * DP_W // 4, DP_TAIL // 4), :])


def _format_table(emb):
    mesh = plsc.VectorSubcoreMesh(core_axis_name="c", subcore_axis_name="s")
    sc = [pltpu.VMEM((EMBED_DIM, DP_W), jnp.float32) for _ in range(2)]
    sc += [pltpu.VMEM((DP_W // 4, 128), jnp.float32) for _ in range(2)]
    sc += [pltpu.SemaphoreType.DMA for _ in range(4)]
    return pl.kernel(
        _fmt_body,
        mesh=mesh,
        out_type=jax.ShapeDtypeStruct((VOCAB // 4, 128), jnp.float32),
        scratch_types=sc,
        compiler_params=pltpu.CompilerParams(
            use_tc_tiling_on_sc=True, needs_layout_passes=False),
    )(jnp.transpose(emb),
      emb[DP_NB * DP_W:, :].reshape(DP_TAIL // 4, 128))


@functools.partial(jax.jit, static_argnames=())
def kernel(x, hidden, emb):
    del hidden
    x5 = jnp.transpose(x, (1, 0)).reshape(SEQ // 8, 8, NW, BT)
    x5 = jnp.transpose(x5, (0, 2, 1, 3))

    embl = _format_table(emb).reshape(VOCAB, EMBED_DIM)

    mesh = plsc.VectorSubcoreMesh(core_axis_name="c", subcore_axis_name="s")
    y3 = pl.kernel(
        _body,
        mesh=mesh,
        out_type=jax.ShapeDtypeStruct((SEQ, DHI, NW * INNER), jnp.float32),
        scratch_types=[
            pltpu.VMEM((SEQ // 8, 8, BT), jnp.int32),
            pltpu.VMEM((LG, BT, EMBED_DIM), jnp.float32),
            pltpu.VMEM((LG, BT, EMBED_DIM), jnp.float32),
            pltpu.VMEM((LG, BT, EMBED_DIM), jnp.float32),
            pltpu.VMEM((LG, DHI, INNER), jnp.float32),
            pltpu.VMEM((LG, DHI, INNER), jnp.float32),
            pltpu.SemaphoreType.DMA,
            pltpu.SemaphoreType.DMA,
            pltpu.SemaphoreType.DMA,
            pltpu.SemaphoreType.DMA,
            pltpu.SemaphoreType.DMA,
        ],
        compiler_params=pltpu.CompilerParams(
            use_tc_tiling_on_sc=False, needs_layout_passes=False),
    )(x5, embl)

    y5 = y3.reshape(SEQ, DHI, NW, 8, BT)
    return jnp.transpose(y5, (0, 2, 4, 1, 3)).reshape(SEQ, BATCH, EMBED_DIM)
